# baseline (speedup 1.0000x reference)
import functools

import jax
import jax.numpy as jnp
from jax import lax
from jax.experimental import pallas as pl
from jax.experimental.pallas import tpu as pltpu
from jax.experimental.pallas import tpu_sc as plsc

_HALFKP = 64 * 5 * 64
_L1 = 256
_L2 = 31
_L3 = 32
_NB = 8
_NP = 8
_D = _L1 + _NP
_A = 30

_G2 = 128


def _seg_sum_sc(flat_idx_t, table, zeros, num_segments, tiled):
    d = table.shape[1]
    info = plsc.get_sparse_core_info()
    nc, ns = info.num_cores, info.num_subcores
    nw = nc * ns
    seg_per_w = num_segments // nw
    idx_per_w = seg_per_w * _A
    nch = seg_per_w // _G2

    mesh = plsc.VectorSubcoreMesh(core_axis_name="c", subcore_axis_name="s")

    @functools.partial(
        pl.kernel,
        mesh=mesh,
        compiler_params=pltpu.CompilerParams(use_tc_tiling_on_sc=tiled),
        out_type=jax.ShapeDtypeStruct((num_segments, d), jnp.float32),
        scratch_types=[
            pltpu.VMEM((idx_per_w,), jnp.int32),
            pltpu.VMEM((3, _G2, d), jnp.float32),
            pltpu.SemaphoreType.DMA,
            pltpu.SemaphoreType.DMA,
        ],
    )
    def k(idx_hbm, table_hbm, zeros_hbm, out_hbm, idx_v, acc_v, sem_g, sem_o):
        wid = lax.axis_index("s") * nc + lax.axis_index("c")
        seg_base = wid * seg_per_w

        def issue_adds(c, p):
            for r in range(_A):
                pltpu.async_copy(
                    table_hbm.at[idx_v.at[pl.ds((c * _A + r) * _G2, _G2)]],
                    acc_v.at[p], sem_g, add=True)

        def drain_adds():
            for _ in range(_A):
                pltpu.make_async_copy(
                    table_hbm.at[pl.ds(0, _G2)], acc_v.at[0], sem_g).wait()

        def drain_out():
            pltpu.make_async_copy(
                table_hbm.at[pl.ds(0, _G2)], acc_v.at[0], sem_o).wait()

        pltpu.sync_copy(
            idx_hbm.at[pl.ds(wid * idx_per_w, idx_per_w)], idx_v)
        pltpu.sync_copy(zeros_hbm, acc_v.at[0])
        issue_adds(0, 0)
        pltpu.sync_copy(zeros_hbm, acc_v.at[1])
        issue_adds(1, 1)

        def chunk_body(c, carry):
            p = lax.rem(c, 3)
            drain_adds()
            @pl.when(c >= 1)
            def _():
                drain_out()
            pltpu.async_copy(
                acc_v.at[p], out_hbm.at[pl.ds(seg_base + c * _G2, _G2)], sem_o)
            @pl.when(c + 2 < nch)
            def _():
                pz = lax.rem(c + 2, 3)
                pltpu.sync_copy(zeros_hbm, acc_v.at[pz])
                issue_adds(c + 2, pz)
            return carry

        lax.fori_loop(0, nch, chunk_body, 0)
        drain_out()

    return k(flat_idx_t, table, zeros)


def _dense_body(acc_ref, ps_ref, us_ref, them_ref, pc_ref, mwT_ref, mb_ref,
                l2wT_ref, l2b_ref, owT_ref, ob_ref, out_ref):
    w_m = acc_ref[0]
    b_m = acc_ref[1]
    wps = ps_ref[0]
    bps = ps_ref[1]
    us = us_ref[...]
    them = them_ref[...]
    pc = pc_ref[...]

    l0a = jnp.clip(us * w_m + them * b_m, 0.0, 1.0)
    l0b = jnp.clip(us * b_m + them * w_m, 0.0, 1.0)
    h = _L1 // 2
    p = jnp.concatenate(
        [l0a[:, :h] * l0a[:, h:], l0b[:, :h] * l0b[:, h:]], axis=1
    ) * (127.0 / 128.0)

    y1 = jnp.dot(p, mwT_ref[...], preferred_element_type=jnp.float32) + mb_ref[...]

    ls = jnp.clip((pc - 1) // 4, 0, 7)

    npb = _L2 + 1
    sel1 = jnp.zeros((y1.shape[0], npb), jnp.float32)
    for k in range(_NB):
        sel1 = sel1 + jnp.where(ls == k, y1[:, k * npb:(k + 1) * npb], 0.0)
    l1x = sel1[:, :_L2]
    l1x_out = sel1[:, _L2:]
    l1cat = jnp.clip(
        jnp.concatenate([l1x * l1x * (255.0 / 256.0), l1x], axis=1), 0.0, 1.0
    )

    y2 = jnp.dot(l1cat, l2wT_ref[...], preferred_element_type=jnp.float32) + l2b_ref[...]
    sel2 = jnp.zeros((y2.shape[0], _L3), jnp.float32)
    for k in range(_NB):
        sel2 = sel2 + jnp.where(ls == k, y2[:, k * _L3:(k + 1) * _L3], 0.0)
    l2x = jnp.clip(sel2, 0.0, 1.0)

    y3 = jnp.dot(l2x, owT_ref[...], preferred_element_type=jnp.float32) + ob_ref[...]
    sel3 = jnp.zeros((y3.shape[0], 1), jnp.float32)
    wps_sel = jnp.zeros((y3.shape[0], 1), jnp.float32)
    bps_sel = jnp.zeros((y3.shape[0], 1), jnp.float32)
    for k in range(_NB):
        m = ls == k
        sel3 = sel3 + jnp.where(m, y3[:, k:k + 1], 0.0)
        wps_sel = wps_sel + jnp.where(m, wps[:, k:k + 1], 0.0)
        bps_sel = bps_sel + jnp.where(m, bps[:, k:k + 1], 0.0)

    out_ref[...] = sel3 + l1x_out + (wps_sel - bps_sel) * (us - 0.5)


def _dense_tc(acc3, ps3, us2, them2, pc2, mwT, mb, l2wT, l2b, owT, ob):
    B = acc3.shape[1]
    Bb = 1024
    grid = (B // Bb,)
    return pl.pallas_call(
        _dense_body,
        grid=grid,
        in_specs=[
            pl.BlockSpec((2, Bb, _L1), lambda i: (0, i, 0)),
            pl.BlockSpec((2, Bb, _NP), lambda i: (0, i, 0)),
            pl.BlockSpec((Bb, 1), lambda i: (i, 0)),
            pl.BlockSpec((Bb, 1), lambda i: (i, 0)),
            pl.BlockSpec((Bb, 1), lambda i: (i, 0)),
            pl.BlockSpec((_L1, (_L2 + 1) * _NB), lambda i: (0, 0)),
            pl.BlockSpec((1, (_L2 + 1) * _NB), lambda i: (0, 0)),
            pl.BlockSpec((2 * _L2, _L3 * _NB), lambda i: (0, 0)),
            pl.BlockSpec((1, _L3 * _NB), lambda i: (0, 0)),
            pl.BlockSpec((_L3, _NB), lambda i: (0, 0)),
            pl.BlockSpec((1, _NB), lambda i: (0, 0)),
        ],
        out_specs=pl.BlockSpec((Bb, 1), lambda i: (i, 0)),
        out_shape=jax.ShapeDtypeStruct((B, 1), jnp.float32),
    )(acc3, ps3, us2, them2, pc2, mwT, mb, l2wT, l2b, owT, ob)


def kernel(emb, l1_w, l1_b, l1f_w, l1f_b, l2_w, l2_b, out_w, out_b,
           us, them, w_idx, b_idx, piece_count):
    B, A = w_idx.shape
    assert A == _A
    info = plsc.get_sparse_core_info()
    nw = info.num_cores * info.num_subcores

    mwT = (l1_w + jnp.tile(l1f_w, (_NB, 1))).T
    mb = (l1_b + jnp.tile(l1f_b, (_NB,))).reshape(1, -1)
    l2wT = l2_w.T
    l2b = l2_b.reshape(1, -1)
    owT = out_w.T
    ob = out_b.reshape(1, -1)
    us2 = us.reshape(B, 1)
    them2 = them.reshape(B, 1)
    pc2 = piece_count.reshape(B, 1).astype(jnp.int32)

    w32 = w_idx.astype(jnp.int32)
    b32 = b_idx.astype(jnp.int32)
    emb_main = emb[:, :_L1]
    emb_ps = emb[:, _L1:]
    zeros_main = jnp.zeros((_G2, _L1), jnp.float32)
    zeros_ps = jnp.zeros((_G2, _NP), jnp.float32)

    full = jnp.concatenate([w32, b32], axis=0)
    nchp = (2 * B // nw) // _G2
    ps_idx = full.reshape(nw, nchp, _G2, _A).transpose(0, 1, 3, 2).reshape(-1)
    ps = _seg_sum_sc(ps_idx, emb_ps, zeros_ps, 2 * B, tiled=False)
    ps3 = ps.reshape(2, B, _NP)

    ns = 4
    bh = B // ns
    outs = []
    for t in range(ns):
        sl = slice(t * bh, (t + 1) * bh)
        flat = jnp.concatenate([w32[sl], b32[sl]], axis=0).reshape(-1)
        nch = (2 * bh // nw) // _G2
        ft = flat.reshape(nw, nch, _G2, _A).transpose(0, 1, 3, 2).reshape(-1)
        acc = _seg_sum_sc(ft, emb_main, zeros_main, 2 * bh, tiled=False)
        acc3 = acc.reshape(2, bh, _L1)
        outs.append(_dense_tc(acc3, ps3[:, sl], us2[sl], them2[sl], pc2[sl],
                              mwT, mb, l2wT, l2b, owT, ob))
    return jnp.concatenate(outs, axis=0)


# speedup vs baseline: 1.0008x; 1.0008x over previous
import functools

import jax
import jax.numpy as jnp
from jax import lax
from jax.experimental import pallas as pl
from jax.experimental.pallas import tpu as pltpu
from jax.experimental.pallas import tpu_sc as plsc

_HALFKP = 64 * 5 * 64
_L1 = 256
_L2 = 31
_L3 = 32
_NB = 8
_NP = 8
_D = _L1 + _NP
_A = 30

_G2 = 128


def _seg_sum_sc(flat_idx_t, table, zeros, num_segments, tiled):
    d = table.shape[1]
    info = plsc.get_sparse_core_info()
    nc, ns = info.num_cores, info.num_subcores
    nw = nc * ns
    seg_per_w = num_segments // nw
    idx_per_w = seg_per_w * _A
    nch = seg_per_w // _G2

    mesh = plsc.VectorSubcoreMesh(core_axis_name="c", subcore_axis_name="s")

    @functools.partial(
        pl.kernel,
        mesh=mesh,
        compiler_params=pltpu.CompilerParams(use_tc_tiling_on_sc=tiled),
        out_type=jax.ShapeDtypeStruct((num_segments, d), jnp.float32),
        scratch_types=[
            pltpu.VMEM((idx_per_w,), jnp.int32),
            pltpu.VMEM((3, _G2, d), jnp.float32),
            pltpu.SemaphoreType.DMA,
            pltpu.SemaphoreType.DMA,
        ],
    )
    def k(idx_hbm, table_hbm, zeros_hbm, out_hbm, idx_v, acc_v, sem_g, sem_o):
        wid = lax.axis_index("s") * nc + lax.axis_index("c")
        seg_base = wid * seg_per_w

        def issue_adds(c, p):
            for r in range(_A):
                pltpu.async_copy(
                    table_hbm.at[idx_v.at[pl.ds((c * _A + r) * _G2, _G2)]],
                    acc_v.at[p], sem_g, add=True)

        def drain_adds():
            for _ in range(_A):
                pltpu.make_async_copy(
                    table_hbm.at[pl.ds(0, _G2)], acc_v.at[0], sem_g).wait()

        def drain_out():
            pltpu.make_async_copy(
                table_hbm.at[pl.ds(0, _G2)], acc_v.at[0], sem_o).wait()

        pltpu.sync_copy(
            idx_hbm.at[pl.ds(wid * idx_per_w, idx_per_w)], idx_v)
        pltpu.sync_copy(zeros_hbm, acc_v.at[0])
        issue_adds(0, 0)
        pltpu.sync_copy(zeros_hbm, acc_v.at[1])
        issue_adds(1, 1)

        def chunk_body(c, carry):
            p = lax.rem(c, 3)
            drain_adds()
            @pl.when(c >= 1)
            def _():
                drain_out()
            pltpu.async_copy(
                acc_v.at[p], out_hbm.at[pl.ds(seg_base + c * _G2, _G2)], sem_o)
            @pl.when(c + 2 < nch)
            def _():
                pz = lax.rem(c + 2, 3)
                pltpu.sync_copy(zeros_hbm, acc_v.at[pz])
                issue_adds(c + 2, pz)
            return carry

        lax.fori_loop(0, nch, chunk_body, 0)
        drain_out()

    return k(flat_idx_t, table, zeros)


def _dense_body(acc_ref, ps_ref, us_ref, them_ref, pc_ref, mwT_ref, mb_ref,
                l2wT_ref, l2b_ref, owT_ref, ob_ref, out_ref):
    w_m = acc_ref[0]
    b_m = acc_ref[1]
    wps = ps_ref[0]
    bps = ps_ref[1]
    us = us_ref[...]
    them = them_ref[...]
    pc = pc_ref[...]

    l0a = jnp.clip(us * w_m + them * b_m, 0.0, 1.0)
    l0b = jnp.clip(us * b_m + them * w_m, 0.0, 1.0)
    h = _L1 // 2
    p = jnp.concatenate(
        [l0a[:, :h] * l0a[:, h:], l0b[:, :h] * l0b[:, h:]], axis=1
    ) * (127.0 / 128.0)

    y1 = jnp.dot(p, mwT_ref[...], preferred_element_type=jnp.float32) + mb_ref[...]

    ls = jnp.clip((pc - 1) // 4, 0, 7)

    npb = _L2 + 1
    sel1 = jnp.zeros((y1.shape[0], npb), jnp.float32)
    for k in range(_NB):
        sel1 = sel1 + jnp.where(ls == k, y1[:, k * npb:(k + 1) * npb], 0.0)
    l1x = sel1[:, :_L2]
    l1x_out = sel1[:, _L2:]
    l1cat = jnp.clip(
        jnp.concatenate([l1x * l1x * (255.0 / 256.0), l1x], axis=1), 0.0, 1.0
    )

    y2 = jnp.dot(l1cat, l2wT_ref[...], preferred_element_type=jnp.float32) + l2b_ref[...]
    sel2 = jnp.zeros((y2.shape[0], _L3), jnp.float32)
    for k in range(_NB):
        sel2 = sel2 + jnp.where(ls == k, y2[:, k * _L3:(k + 1) * _L3], 0.0)
    l2x = jnp.clip(sel2, 0.0, 1.0)

    y3 = jnp.dot(l2x, owT_ref[...], preferred_element_type=jnp.float32) + ob_ref[...]
    sel3 = jnp.zeros((y3.shape[0], 1), jnp.float32)
    wps_sel = jnp.zeros((y3.shape[0], 1), jnp.float32)
    bps_sel = jnp.zeros((y3.shape[0], 1), jnp.float32)
    for k in range(_NB):
        m = ls == k
        sel3 = sel3 + jnp.where(m, y3[:, k:k + 1], 0.0)
        wps_sel = wps_sel + jnp.where(m, wps[:, k:k + 1], 0.0)
        bps_sel = bps_sel + jnp.where(m, bps[:, k:k + 1], 0.0)

    out_ref[...] = sel3 + l1x_out + (wps_sel - bps_sel) * (us - 0.5)


def _dense_tc(acc3, ps3, us2, them2, pc2, mwT, mb, l2wT, l2b, owT, ob):
    B = acc3.shape[1]
    Bb = 1024
    grid = (B // Bb,)
    return pl.pallas_call(
        _dense_body,
        grid=grid,
        in_specs=[
            pl.BlockSpec((2, Bb, _L1), lambda i: (0, i, 0)),
            pl.BlockSpec((2, Bb, _NP), lambda i: (0, i, 0)),
            pl.BlockSpec((Bb, 1), lambda i: (i, 0)),
            pl.BlockSpec((Bb, 1), lambda i: (i, 0)),
            pl.BlockSpec((Bb, 1), lambda i: (i, 0)),
            pl.BlockSpec((_L1, (_L2 + 1) * _NB), lambda i: (0, 0)),
            pl.BlockSpec((1, (_L2 + 1) * _NB), lambda i: (0, 0)),
            pl.BlockSpec((2 * _L2, _L3 * _NB), lambda i: (0, 0)),
            pl.BlockSpec((1, _L3 * _NB), lambda i: (0, 0)),
            pl.BlockSpec((_L3, _NB), lambda i: (0, 0)),
            pl.BlockSpec((1, _NB), lambda i: (0, 0)),
        ],
        out_specs=pl.BlockSpec((Bb, 1), lambda i: (i, 0)),
        out_shape=jax.ShapeDtypeStruct((B, 1), jnp.float32),
    )(acc3, ps3, us2, them2, pc2, mwT, mb, l2wT, l2b, owT, ob)


def kernel(emb, l1_w, l1_b, l1f_w, l1f_b, l2_w, l2_b, out_w, out_b,
           us, them, w_idx, b_idx, piece_count):
    B, A = w_idx.shape
    assert A == _A
    info = plsc.get_sparse_core_info()
    nw = info.num_cores * info.num_subcores

    mwT = (l1_w + jnp.tile(l1f_w, (_NB, 1))).T
    mb = (l1_b + jnp.tile(l1f_b, (_NB,))).reshape(1, -1)
    l2wT = l2_w.T
    l2b = l2_b.reshape(1, -1)
    owT = out_w.T
    ob = out_b.reshape(1, -1)
    us2 = us.reshape(B, 1)
    them2 = them.reshape(B, 1)
    pc2 = piece_count.reshape(B, 1).astype(jnp.int32)

    w32 = w_idx.astype(jnp.int32)
    b32 = b_idx.astype(jnp.int32)
    emb_main = emb[:, :_L1]
    emb_ps = emb[:, _L1:]
    zeros_main = jnp.zeros((_G2, _L1), jnp.float32)
    zeros_ps = jnp.zeros((_G2, _NP), jnp.float32)

    ns = 4
    bh = B // ns
    accs = []
    for t in range(ns):
        sl = slice(t * bh, (t + 1) * bh)
        flat = jnp.concatenate([w32[sl], b32[sl]], axis=0).reshape(-1)
        nch = (2 * bh // nw) // _G2
        ft = flat.reshape(nw, nch, _G2, _A).transpose(0, 1, 3, 2).reshape(-1)
        accs.append(_seg_sum_sc(ft, emb_main, zeros_main, 2 * bh, tiled=False))
        if t == 0:
            full = jnp.concatenate([w32, b32], axis=0)
            nchp = (2 * B // nw) // _G2
            ps_idx = full.reshape(nw, nchp, _G2, _A).transpose(0, 1, 3, 2).reshape(-1)
            ps = _seg_sum_sc(ps_idx, emb_ps, zeros_ps, 2 * B, tiled=False)
            ps3 = ps.reshape(2, B, _NP)
    outs = []
    for t in range(ns):
        sl = slice(t * bh, (t + 1) * bh)
        acc3 = accs[t].reshape(2, bh, _L1)
        outs.append(_dense_tc(acc3, ps3[:, sl], us2[sl], them2[sl], pc2[sl],
                              mwT, mb, l2wT, l2b, owT, ob))
    return jnp.concatenate(outs, axis=0)
